# baseline (speedup 1.0000x reference)
import functools

import jax
import jax.numpy as jnp
from jax import lax
from jax.experimental import pallas as pl
from jax.experimental.pallas import tpu as pltpu
from jax.experimental.pallas import tpu_sc as plsc

NSTU = 8192
NTEA = 8192
HALF = NSTU // 2
NCLS = 22
CPAD = 32
SDIM = 64
HDIM = 32
TDIM = 512
TEMP = 2.0


KNN_BS = 1024
KNN_BT = 1024


def _knn_body(q_ref, kt_ref, idx_ref):
    kt = kt_ref[...]
    k2 = jnp.sum(kt * kt, axis=0, keepdims=True)
    row8 = lax.broadcasted_iota(jnp.int32, (8, NTEA), 0)
    kta = jnp.where(row8 == 3, k2, jnp.where(row8 == 4, 1.0, kt))
    qt = jnp.pad(q_ref[...], ((0, 5), (0, 0)))
    q2 = jnp.sum(qt * qt, axis=0, keepdims=True)
    row8b = lax.broadcasted_iota(jnp.int32, (8, KNN_BS), 0)
    qat = jnp.where(row8b == 3, 1.0, jnp.where(row8b == 4, q2, qt * (-2.0)))
    qa = lax.transpose(qat, (1, 0))
    run_key = jnp.full((KNN_BS,), jnp.inf, jnp.float32)
    run_cid = jnp.zeros((KNN_BS,), jnp.int32)
    for c in range(NTEA // KNN_BT):
        s = lax.dot_general(qa, kta[:, c * KNN_BT:(c + 1) * KNN_BT],
                            (((1,), (0,)), ((), ())),
                            preferred_element_type=jnp.float32)
        si = lax.bitcast_convert_type(s, jnp.int32)
        li = lax.broadcasted_iota(jnp.int32, (KNN_BS, KNN_BT), 1)
        key = lax.bitcast_convert_type((si & -KNN_BT) | li, jnp.float32)
        mk = jnp.min(key, axis=1)
        upd = mk < run_key
        run_key = jnp.where(upd, mk, run_key)
        run_cid = jnp.where(upd, c, run_cid)
    local = lax.bitcast_convert_type(run_key, jnp.int32) & (KNN_BT - 1)
    idx_ref[...] = run_cid * KNN_BT + local


def _knn_call(sc, ktp, half):
    base = half * (HALF // KNN_BS)
    return pl.pallas_call(
        _knn_body,
        grid=(HALF // KNN_BS,),
        in_specs=[
            pl.BlockSpec((3, KNN_BS), lambda i: (0, i + base)),
            pl.BlockSpec((8, NTEA), lambda i: (0, 0)),
        ],
        out_specs=pl.BlockSpec((KNN_BS,), lambda i: (i,)),
        out_shape=jax.ShapeDtypeStruct((HALF,), jnp.int32),
    )(sc, ktp)


SC_NW = 32
SC_BPW = HALF // SC_NW
SC_ROWS = 64
SC_NCH = SC_BPW // SC_ROWS


def _gather_call(tlp, tf, idx):
    mesh = plsc.VectorSubcoreMesh(core_axis_name="c", subcore_axis_name="s")

    @functools.partial(
        pl.kernel,
        out_type=[jax.ShapeDtypeStruct((HALF, CPAD), jnp.float32),
                  jax.ShapeDtypeStruct((HALF, TDIM), jnp.float32)],
        mesh=mesh,
        scratch_types=[
            pltpu.VMEM((SC_BPW,), jnp.int32),
            pltpu.VMEM((SC_NCH, SC_ROWS), jnp.int32),
            pltpu.VMEM((SC_BPW, CPAD), jnp.float32),
            pltpu.VMEM((SC_ROWS, TDIM), jnp.float32),
            pltpu.VMEM((SC_ROWS, TDIM), jnp.float32),
            pltpu.SemaphoreType.DMA,
            pltpu.SemaphoreType.DMA,
            pltpu.SemaphoreType.DMA,
        ],
        compiler_params=pltpu.CompilerParams(use_tc_tiling_on_sc=False),
    )
    def gk(tl_hbm, tf_hbm, idx_hbm, out_l, out_f,
           idx_v, idx2, lrows, fbuf0, fbuf1, sem0, sem1, seml):
        wid = lax.axis_index("s") * 2 + lax.axis_index("c")
        base = wid * SC_BPW
        pltpu.sync_copy(idx_hbm.at[pl.ds(base, SC_BPW)], idx_v)
        for c in range(SC_NCH):
            pltpu.sync_copy(idx_hbm.at[pl.ds(base + c * SC_ROWS, SC_ROWS)],
                            idx2.at[c])
        bufs = [fbuf0, fbuf1]
        sems = [sem0, sem1]
        cl = pltpu.async_copy(tl_hbm.at[idx_v], lrows, seml)
        pend = [pltpu.async_copy(tf_hbm.at[idx2.at[0]], bufs[0], sems[0]),
                pltpu.async_copy(tf_hbm.at[idx2.at[1]], bufs[1], sems[1])]
        for c in range(SC_NCH):
            b = c % 2
            pend[b].wait()
            pltpu.sync_copy(bufs[b],
                            out_f.at[pl.ds(base + c * SC_ROWS, SC_ROWS)])
            if c + 2 < SC_NCH:
                pend[b] = pltpu.async_copy(tf_hbm.at[idx2.at[c + 2]],
                                           bufs[b], sems[b])
        cl.wait()
        pltpu.sync_copy(lrows, out_l.at[pl.ds(base, SC_BPW)])

    return gk(tlp, tf, idx)


LOSS_BS = 1024


def _loss_body(is_final, *refs):
    if is_final:
        (seg_ref, ml_ref, sf_ref, mf_ref, w1t_ref, b1_ref, w2t_ref, b2_ref,
         lw_ref, part_ref, out_ref, acc_ref) = refs
    else:
        (seg_ref, ml_ref, sf_ref, mf_ref, w1t_ref, b1_ref, w2t_ref, b2_ref,
         out_ref, acc_ref) = refs
    i = pl.program_id(0)
    nblk = pl.num_programs(0)

    @pl.when(i == 0)
    def _():
        acc_ref[0] = 0.0
        acc_ref[1] = 0.0

    segt = seg_ref[...] * (1.0 / TEMP)
    smax = jnp.max(segt, axis=0, keepdims=True)
    se = jnp.exp(segt - smax)
    ssum = jnp.sum(se, axis=0, keepdims=True)
    slogpt = segt - smax - jnp.log(ssum)
    mlg = ml_ref[:, :NCLS] * (1.0 / TEMP)
    tmax = jnp.max(mlg, axis=1, keepdims=True)
    te = jnp.exp(mlg - tmax)
    tsum = jnp.sum(te, axis=1, keepdims=True)
    tp = te / tsum
    tlogp = mlg - tmax - jnp.log(tsum)
    cc = lax.dot_general(slogpt, tp, (((1,), (0,)), ((), ())),
                         preferred_element_type=jnp.float32)
    diag = (lax.broadcasted_iota(jnp.int32, (NCLS, NCLS), 0)
            == lax.broadcasted_iota(jnp.int32, (NCLS, NCLS), 1))
    kl_blk = jnp.sum(tp * tlogp) - jnp.sum(jnp.where(diag, cc, 0.0))

    ht = lax.dot_general(w1t_ref[...], sf_ref[...], (((1,), (0,)), ((), ())),
                         preferred_element_type=jnp.float32)
    ht = jnp.maximum(ht + b1_ref[...], 0.0)
    h = lax.transpose(ht, (1, 0))
    proj = lax.dot_general(h, w2t_ref[...], (((1,), (1,)), ((), ())),
                           preferred_element_type=jnp.float32)
    proj = proj + b2_ref[...][None, :]
    mf = mf_ref[...]
    pn = jnp.sqrt(jnp.sum(proj * proj, axis=1, keepdims=True))
    sn = proj / jnp.maximum(pn, 1e-12)
    tn2 = jnp.sqrt(jnp.sum(mf * mf, axis=1, keepdims=True))
    tn = mf / jnp.maximum(tn2, 1e-12)
    dd = sn - tn
    ft_blk = jnp.sum(dd * dd)

    kl_tot = acc_ref[0] + kl_blk
    ft_tot = acc_ref[1] + ft_blk
    acc_ref[0] = kl_tot
    acc_ref[1] = ft_tot

    @pl.when(i == nblk - 1)
    def _():
        if is_final:
            kl_all = kl_tot + part_ref[0]
            ft_all = ft_tot + part_ref[1]
            lw = lw_ref[...]
            we = jnp.exp(lw - jnp.max(lw))
            i5 = lax.broadcasted_iota(jnp.int32, (5,), 0)
            w4 = jnp.sum(jnp.where(i5 == 4, we, 0.0)) / jnp.sum(we)
            kl_loss = 0.5 * (kl_all / NSTU) * (TEMP * TEMP)
            ft_loss = ft_all / (NSTU * TDIM)
            out_ref[...] = jnp.full((1, 1), kl_loss + w4 * ft_loss,
                                    jnp.float32)
        else:
            out_ref[0] = kl_tot
            out_ref[1] = ft_tot


def _half_specs(half):
    base = half * (HALF // LOSS_BS)
    return [
        pl.BlockSpec((NCLS, LOSS_BS), lambda i: (0, i + base)),
        pl.BlockSpec((LOSS_BS, CPAD), lambda i: (i, 0)),
        pl.BlockSpec((SDIM, LOSS_BS), lambda i: (0, i + base)),
        pl.BlockSpec((LOSS_BS, TDIM), lambda i: (i, 0)),
        pl.BlockSpec((HDIM, SDIM), lambda i: (0, 0)),
        pl.BlockSpec((HDIM, 1), lambda i: (0, 0)),
        pl.BlockSpec((TDIM, HDIM), lambda i: (0, 0)),
        pl.BlockSpec((TDIM,), lambda i: (0,)),
    ]


def _loss_part_call(seg, ml, sf, mf, w1, b1, w2, b2):
    return pl.pallas_call(
        functools.partial(_loss_body, False),
        grid=(HALF // LOSS_BS,),
        in_specs=_half_specs(0),
        out_specs=pl.BlockSpec(memory_space=pltpu.SMEM),
        out_shape=jax.ShapeDtypeStruct((2,), jnp.float32),
        scratch_shapes=[pltpu.SMEM((2,), jnp.float32)],
    )(seg, ml, sf, mf, w1, b1, w2, b2)


def _loss_final_call(seg, ml, sf, mf, w1, b1, w2, b2, lw, part):
    return pl.pallas_call(
        functools.partial(_loss_body, True),
        grid=(HALF // LOSS_BS,),
        in_specs=_half_specs(1) + [
            pl.BlockSpec((5,), lambda i: (0,)),
            pl.BlockSpec(memory_space=pltpu.SMEM),
        ],
        out_specs=pl.BlockSpec((1, 1), lambda i: (0, 0)),
        out_shape=jax.ShapeDtypeStruct((1, 1), jnp.float32),
        scratch_shapes=[pltpu.SMEM((2,), jnp.float32)],
    )(seg, ml, sf, mf, w1, b1, w2, b2, lw, part)


def kernel(student_coords, teacher_coords, teacher_logits, seg_logits,
           student_feat, teacher_feat, W1, b1, W2, b2, layer_weight_params):
    ktp = jnp.pad(teacher_coords, ((0, 0), (0, 5))).T
    tlp = jnp.pad(teacher_logits, ((0, 0), (0, CPAD - NCLS)))
    sct = student_coords.T
    segt = seg_logits.T
    sft = student_feat.T
    w1t = W1.T
    w2t = W2.T
    b1r = b1.reshape(HDIM, 1)
    idx0 = _knn_call(sct, ktp, 0)
    ml0, mf0 = _gather_call(tlp, teacher_feat, idx0)
    idx1 = _knn_call(sct, ktp, 1)
    ml1, mf1 = _gather_call(tlp, teacher_feat, idx1)
    part = _loss_part_call(segt, ml0, sft, mf0, w1t, b1r, w2t, b2)
    out = _loss_final_call(segt, ml1, sft, mf1, w1t, b1r, w2t, b2,
                           layer_weight_params, part)
    return out[0, 0]


# speedup vs baseline: 1.3483x; 1.3483x over previous
import functools

import jax
import jax.numpy as jnp
from jax import lax
from jax.experimental import pallas as pl
from jax.experimental.pallas import tpu as pltpu
from jax.experimental.pallas import tpu_sc as plsc

NSTU = 8192
NTEA = 8192
HALF = NSTU // 2
NCLS = 22
CPAD = 128
SDIM = 64
HDIM = 32
TDIM = 512
TEMP = 2.0


KNN_BS = 1024
KNN_BT = 1024


def _knn_body(q_ref, kt_ref, idx_ref):
    kt = kt_ref[...]
    k2 = jnp.sum(kt * kt, axis=0, keepdims=True)
    row8 = lax.broadcasted_iota(jnp.int32, (8, NTEA), 0)
    kta = jnp.where(row8 == 3, k2, jnp.where(row8 == 4, 1.0, kt))
    qt = jnp.pad(q_ref[...], ((0, 5), (0, 0)))
    q2 = jnp.sum(qt * qt, axis=0, keepdims=True)
    row8b = lax.broadcasted_iota(jnp.int32, (8, KNN_BS), 0)
    qat = jnp.where(row8b == 3, 1.0, jnp.where(row8b == 4, q2, qt * (-2.0)))
    qa = lax.transpose(qat, (1, 0))
    run_key = jnp.full((KNN_BS,), jnp.inf, jnp.float32)
    run_cid = jnp.zeros((KNN_BS,), jnp.int32)
    for c in range(NTEA // KNN_BT):
        s = lax.dot_general(qa, kta[:, c * KNN_BT:(c + 1) * KNN_BT],
                            (((1,), (0,)), ((), ())),
                            preferred_element_type=jnp.float32)
        si = lax.bitcast_convert_type(s, jnp.int32)
        li = lax.broadcasted_iota(jnp.int32, (KNN_BS, KNN_BT), 1)
        key = lax.bitcast_convert_type((si & -KNN_BT) | li, jnp.float32)
        mk = jnp.min(key, axis=1)
        upd = mk < run_key
        run_key = jnp.where(upd, mk, run_key)
        run_cid = jnp.where(upd, c, run_cid)
    local = lax.bitcast_convert_type(run_key, jnp.int32) & (KNN_BT - 1)
    idx_ref[...] = run_cid * KNN_BT + local


def _knn_call(sc, ktp, half):
    base = half * (HALF // KNN_BS)
    return pl.pallas_call(
        _knn_body,
        grid=(HALF // KNN_BS,),
        in_specs=[
            pl.BlockSpec((3, KNN_BS), lambda i: (0, i + base)),
            pl.BlockSpec((8, NTEA), lambda i: (0, 0)),
        ],
        out_specs=pl.BlockSpec((KNN_BS,), lambda i: (i,)),
        out_shape=jax.ShapeDtypeStruct((HALF,), jnp.int32),
    )(sc, ktp)


SC_NW = 32
SC_BPW = HALF // SC_NW
SC_ROWS = 64
SC_NCH = SC_BPW // SC_ROWS


def _gather_call(tlp, tf, idx):
    mesh = plsc.VectorSubcoreMesh(core_axis_name="c", subcore_axis_name="s")

    @functools.partial(
        pl.kernel,
        out_type=[jax.ShapeDtypeStruct((HALF, CPAD), jnp.float32),
                  jax.ShapeDtypeStruct((HALF, TDIM), jnp.float32)],
        mesh=mesh,
        scratch_types=[
            pltpu.VMEM((SC_BPW,), jnp.int32),
            pltpu.VMEM((SC_NCH, SC_ROWS), jnp.int32),
            pltpu.VMEM((SC_BPW, CPAD), jnp.float32),
            pltpu.VMEM((SC_ROWS, TDIM), jnp.float32),
            pltpu.VMEM((SC_ROWS, TDIM), jnp.float32),
            pltpu.SemaphoreType.DMA,
            pltpu.SemaphoreType.DMA,
            pltpu.SemaphoreType.DMA,
        ],
    )
    def gk(tl_hbm, tf_hbm, idx_hbm, out_l, out_f,
           idx_v, idx2, lrows, fbuf0, fbuf1, sem0, sem1, seml):
        wid = lax.axis_index("s") * 2 + lax.axis_index("c")
        base = wid * SC_BPW
        pltpu.sync_copy(idx_hbm.at[pl.ds(base, SC_BPW)], idx_v)
        for c in range(SC_NCH):
            pltpu.sync_copy(idx_hbm.at[pl.ds(base + c * SC_ROWS, SC_ROWS)],
                            idx2.at[c])
        bufs = [fbuf0, fbuf1]
        sems = [sem0, sem1]
        cl = pltpu.async_copy(tl_hbm.at[idx_v], lrows, seml)
        pend = [pltpu.async_copy(tf_hbm.at[idx2.at[0]], bufs[0], sems[0]),
                pltpu.async_copy(tf_hbm.at[idx2.at[1]], bufs[1], sems[1])]
        for c in range(SC_NCH):
            b = c % 2
            pend[b].wait()
            pltpu.sync_copy(bufs[b],
                            out_f.at[pl.ds(base + c * SC_ROWS, SC_ROWS)])
            if c + 2 < SC_NCH:
                pend[b] = pltpu.async_copy(tf_hbm.at[idx2.at[c + 2]],
                                           bufs[b], sems[b])
        cl.wait()
        pltpu.sync_copy(lrows, out_l.at[pl.ds(base, SC_BPW)])

    return gk(tlp, tf, idx)


LOSS_BS = 1024


def _loss_body(is_final, *refs):
    if is_final:
        (seg_ref, ml_ref, sf_ref, mf_ref, w1t_ref, b1_ref, w2t_ref, b2_ref,
         lw_ref, part_ref, out_ref, acc_ref) = refs
    else:
        (seg_ref, ml_ref, sf_ref, mf_ref, w1t_ref, b1_ref, w2t_ref, b2_ref,
         out_ref, acc_ref) = refs
    i = pl.program_id(0)
    nblk = pl.num_programs(0)

    @pl.when(i == 0)
    def _():
        acc_ref[0] = 0.0
        acc_ref[1] = 0.0

    segt = seg_ref[...] * (1.0 / TEMP)
    smax = jnp.max(segt, axis=0, keepdims=True)
    se = jnp.exp(segt - smax)
    ssum = jnp.sum(se, axis=0, keepdims=True)
    slogpt = segt - smax - jnp.log(ssum)
    mlg = ml_ref[:, :NCLS] * (1.0 / TEMP)
    tmax = jnp.max(mlg, axis=1, keepdims=True)
    te = jnp.exp(mlg - tmax)
    tsum = jnp.sum(te, axis=1, keepdims=True)
    tp = te / tsum
    tlogp = mlg - tmax - jnp.log(tsum)
    cc = lax.dot_general(slogpt, tp, (((1,), (0,)), ((), ())),
                         preferred_element_type=jnp.float32)
    diag = (lax.broadcasted_iota(jnp.int32, (NCLS, NCLS), 0)
            == lax.broadcasted_iota(jnp.int32, (NCLS, NCLS), 1))
    kl_blk = jnp.sum(tp * tlogp) - jnp.sum(jnp.where(diag, cc, 0.0))

    ht = lax.dot_general(w1t_ref[...], sf_ref[...], (((1,), (0,)), ((), ())),
                         preferred_element_type=jnp.float32)
    ht = jnp.maximum(ht + b1_ref[...], 0.0)
    h = lax.transpose(ht, (1, 0))
    proj = lax.dot_general(h, w2t_ref[...], (((1,), (1,)), ((), ())),
                           preferred_element_type=jnp.float32)
    proj = proj + b2_ref[...][None, :]
    mf = mf_ref[...]
    pn = jnp.sqrt(jnp.sum(proj * proj, axis=1, keepdims=True))
    sn = proj / jnp.maximum(pn, 1e-12)
    tn2 = jnp.sqrt(jnp.sum(mf * mf, axis=1, keepdims=True))
    tn = mf / jnp.maximum(tn2, 1e-12)
    dd = sn - tn
    ft_blk = jnp.sum(dd * dd)

    kl_tot = acc_ref[0] + kl_blk
    ft_tot = acc_ref[1] + ft_blk
    acc_ref[0] = kl_tot
    acc_ref[1] = ft_tot

    @pl.when(i == nblk - 1)
    def _():
        if is_final:
            kl_all = kl_tot + part_ref[0]
            ft_all = ft_tot + part_ref[1]
            lw = lw_ref[...]
            we = jnp.exp(lw - jnp.max(lw))
            i5 = lax.broadcasted_iota(jnp.int32, (5,), 0)
            w4 = jnp.sum(jnp.where(i5 == 4, we, 0.0)) / jnp.sum(we)
            kl_loss = 0.5 * (kl_all / NSTU) * (TEMP * TEMP)
            ft_loss = ft_all / (NSTU * TDIM)
            out_ref[...] = jnp.full((1, 1), kl_loss + w4 * ft_loss,
                                    jnp.float32)
        else:
            out_ref[0] = kl_tot
            out_ref[1] = ft_tot


def _half_specs(half):
    base = half * (HALF // LOSS_BS)
    return [
        pl.BlockSpec((NCLS, LOSS_BS), lambda i: (0, i + base)),
        pl.BlockSpec((LOSS_BS, CPAD), lambda i: (i, 0)),
        pl.BlockSpec((SDIM, LOSS_BS), lambda i: (0, i + base)),
        pl.BlockSpec((LOSS_BS, TDIM), lambda i: (i, 0)),
        pl.BlockSpec((HDIM, SDIM), lambda i: (0, 0)),
        pl.BlockSpec((HDIM, 1), lambda i: (0, 0)),
        pl.BlockSpec((TDIM, HDIM), lambda i: (0, 0)),
        pl.BlockSpec((TDIM,), lambda i: (0,)),
    ]


def _loss_part_call(seg, ml, sf, mf, w1, b1, w2, b2):
    return pl.pallas_call(
        functools.partial(_loss_body, False),
        grid=(HALF // LOSS_BS,),
        in_specs=_half_specs(0),
        out_specs=pl.BlockSpec(memory_space=pltpu.SMEM),
        out_shape=jax.ShapeDtypeStruct((2,), jnp.float32),
        scratch_shapes=[pltpu.SMEM((2,), jnp.float32)],
    )(seg, ml, sf, mf, w1, b1, w2, b2)


def _loss_final_call(seg, ml, sf, mf, w1, b1, w2, b2, lw, part):
    return pl.pallas_call(
        functools.partial(_loss_body, True),
        grid=(HALF // LOSS_BS,),
        in_specs=_half_specs(1) + [
            pl.BlockSpec((5,), lambda i: (0,)),
            pl.BlockSpec(memory_space=pltpu.SMEM),
        ],
        out_specs=pl.BlockSpec((1, 1), lambda i: (0, 0)),
        out_shape=jax.ShapeDtypeStruct((1, 1), jnp.float32),
        scratch_shapes=[pltpu.SMEM((2,), jnp.float32)],
    )(seg, ml, sf, mf, w1, b1, w2, b2, lw, part)


def kernel(student_coords, teacher_coords, teacher_logits, seg_logits,
           student_feat, teacher_feat, W1, b1, W2, b2, layer_weight_params):
    ktp = jnp.pad(teacher_coords, ((0, 0), (0, 5))).T
    tlp = jnp.pad(teacher_logits, ((0, 0), (0, CPAD - NCLS)))
    sct = student_coords.T
    segt = seg_logits.T
    sft = student_feat.T
    w1t = W1.T
    w2t = W2.T
    b1r = b1.reshape(HDIM, 1)
    idx0 = _knn_call(sct, ktp, 0)
    ml0, mf0 = _gather_call(tlp, teacher_feat, idx0)
    idx1 = _knn_call(sct, ktp, 1)
    ml1, mf1 = _gather_call(tlp, teacher_feat, idx1)
    part = _loss_part_call(segt, ml0, sft, mf0, w1t, b1r, w2t, b2)
    out = _loss_final_call(segt, ml1, sft, mf1, w1t, b1r, w2t, b2,
                           layer_weight_params, part)
    return out[0, 0]
